# baseline (speedup 1.0000x reference)
import jax
import jax.numpy as jnp
from jax import lax
from jax.experimental import pallas as pl
from jax.experimental.pallas import tpu as pltpu
from jax.experimental.pallas import tpu_sc as plsc

NC = 2
NS = 16
L = 16
NW = NC * NS

P = 4096 * 50
M = 8
D = 32
PW = P // NW
C = 128
RC = C * M
NCHUNK = PW // C
IDXW = 128
NSTREAM = RC // IDXW
BLOB = 3 * C * M
MASK_OFF = C * M
MT_OFF = 2 * C * M


def _stage(blob_hbm, gc, bufin, sem_in):
    return pltpu.async_copy(blob_hbm.at[gc], bufin, sem_in)


def _fire_gathers(table_hbm, bufin, rows, sem_g):
    for j in range(NSTREAM):
        pltpu.async_copy(table_hbm.at[bufin.at[pl.ds(j * IDXW, IDXW)]],
                         rows.at[pl.ds(j * IDXW, IDXW)], sem_g)


def _drain_gathers(table_hbm, bufin, rows, sem_g):
    for j in range(NSTREAM):
        pltpu.make_async_copy(table_hbm.at[bufin.at[pl.ds(j * IDXW, IDXW)]],
                              rows.at[pl.ds(j * IDXW, IDXW)], sem_g).wait()


def _compute(bufin, rows, out_v):
    def group_body(pg, c):
        cnt = bufin[pl.ds(MT_OFF + pg * L, L)]
        for m in range(1, M):
            cnt = cnt + bufin[pl.ds(MT_OFF + m * C + pg * L, L)]
        a16 = 1.0 / jnp.maximum(cnt.astype(jnp.float32), 1.0)
        for q in range(0, L, 2):
            mm = bufin[pl.ds(MASK_OFF + pg * IDXW + q * M, L)]
            mmf = mm.astype(jnp.float32)
            for t in range(2):
                p = pg * L + q + t
                r = p * M
                lo = rows[r, pl.ds(0, L)] * mmf[t * M]
                hi = rows[r, pl.ds(L, L)] * mmf[t * M]
                for m in range(1, M):
                    lo = lo + rows[r + m, pl.ds(0, L)] * mmf[t * M + m]
                    hi = hi + rows[r + m, pl.ds(L, L)] * mmf[t * M + m]
                a = a16[q + t]
                out_v[p, pl.ds(0, L)] = lo * a
                out_v[p, pl.ds(L, L)] = hi * a
        return c
    lax.fori_loop(0, C // L, group_body, 0)


def _body(blob_hbm, table_hbm, out_hbm,
          bufin0, bufin1, rows0, rows1, outv0, outv1,
          sem_in0, sem_in1, sem_g0, sem_g1, sem_o0, sem_o1):
    w = lax.axis_index("s") * NC + lax.axis_index("c")
    gc0 = w * NCHUNK

    _stage(blob_hbm, gc0, bufin0, sem_in0).wait()
    _fire_gathers(table_hbm, bufin0, rows0, sem_g0)
    _stage(blob_hbm, gc0 + 1, bufin1, sem_in1)

    def pair_body(g, carry):
        ca = gc0 + 2 * g
        cb = ca + 1

        pltpu.make_async_copy(blob_hbm.at[cb], bufin1, sem_in1).wait()
        _fire_gathers(table_hbm, bufin1, rows1, sem_g1)

        _drain_gathers(table_hbm, bufin0, rows0, sem_g0)

        @pl.when(g > 0)
        def _():
            pltpu.make_async_copy(outv0, out_hbm.at[pl.ds(0, C)], sem_o0).wait()
        _compute(bufin0, rows0, outv0)
        pltpu.async_copy(outv0, out_hbm.at[pl.ds(ca * C, C)], sem_o0)

        @pl.when(g < NCHUNK // 2 - 1)
        def _():
            _stage(blob_hbm, ca + 2, bufin0, sem_in0).wait()
            _fire_gathers(table_hbm, bufin0, rows0, sem_g0)

        _drain_gathers(table_hbm, bufin1, rows1, sem_g1)

        @pl.when(g > 0)
        def _():
            pltpu.make_async_copy(outv1, out_hbm.at[pl.ds(0, C)], sem_o1).wait()
        _compute(bufin1, rows1, outv1)
        pltpu.async_copy(outv1, out_hbm.at[pl.ds(cb * C, C)], sem_o1)

        @pl.when(g < NCHUNK // 2 - 1)
        def _():
            _stage(blob_hbm, cb + 2, bufin1, sem_in1)
        return carry

    lax.fori_loop(0, NCHUNK // 2, pair_body, 0)

    pltpu.make_async_copy(outv0, out_hbm.at[pl.ds(0, C)], sem_o0).wait()
    pltpu.make_async_copy(outv1, out_hbm.at[pl.ds(0, C)], sem_o1).wait()


def kernel(category_ids, category_mask, embedding_table):
    nchunks = P // C
    ids_c = category_ids.reshape(nchunks, C * M).astype(jnp.int32)
    mask_pm = category_mask.reshape(nchunks, C * M).astype(jnp.int32)
    mask_mc = (category_mask.reshape(nchunks, C, M).astype(jnp.int32)
               .transpose(0, 2, 1).reshape(nchunks, C * M))
    blob = jnp.concatenate([ids_c, mask_pm, mask_mc], axis=1)

    mesh = plsc.VectorSubcoreMesh(core_axis_name="c", subcore_axis_name="s",
                                  num_cores=NC, num_subcores=NS)
    out = pl.kernel(
        _body,
        out_type=jax.ShapeDtypeStruct((P, D), jnp.float32),
        mesh=mesh,
        compiler_params=pltpu.CompilerParams(use_tc_tiling_on_sc=False),
        scratch_types=[
            pltpu.VMEM((BLOB,), jnp.int32),
            pltpu.VMEM((BLOB,), jnp.int32),
            pltpu.VMEM((RC, D), jnp.float32),
            pltpu.VMEM((RC, D), jnp.float32),
            pltpu.VMEM((C, D), jnp.float32),
            pltpu.VMEM((C, D), jnp.float32),
            pltpu.SemaphoreType.DMA,
            pltpu.SemaphoreType.DMA,
            pltpu.SemaphoreType.DMA,
            pltpu.SemaphoreType.DMA,
            pltpu.SemaphoreType.DMA,
            pltpu.SemaphoreType.DMA,
        ],
    )(blob, embedding_table)
    return out.reshape(category_ids.shape[0], category_ids.shape[1], D)


# speedup vs baseline: 1.0018x; 1.0018x over previous
import jax
import jax.numpy as jnp
from jax import lax
from jax.experimental import pallas as pl
from jax.experimental.pallas import tpu as pltpu
from jax.experimental.pallas import tpu_sc as plsc

NC = 2
NS = 16
L = 16
NW = NC * NS

P = 4096 * 50
M = 8
D = 32
PW = P // NW
C = 128
RC = C * M
NCHUNK = PW // C
IDXW = 128
NSTREAM = RC // IDXW


def _stage(ids_hbm, mask_hbm, gc, bufid, bufmk, sem_in):
    pltpu.async_copy(ids_hbm.at[gc], bufid, sem_in)
    pltpu.async_copy(mask_hbm.at[gc], bufmk, sem_in)


def _stage_wait(ids_hbm, mask_hbm, gc, bufid, bufmk, sem_in):
    pltpu.make_async_copy(ids_hbm.at[gc], bufid, sem_in).wait()
    pltpu.make_async_copy(mask_hbm.at[gc], bufmk, sem_in).wait()


def _fire_gathers(table_hbm, bufid, rows, sem_g):
    for j in range(NSTREAM):
        pltpu.async_copy(table_hbm.at[bufid.at[pl.ds(j * IDXW, IDXW)]],
                         rows.at[pl.ds(j * IDXW, IDXW)], sem_g)


def _drain_gathers(table_hbm, bufid, rows, sem_g):
    for j in range(NSTREAM):
        pltpu.make_async_copy(table_hbm.at[bufid.at[pl.ds(j * IDXW, IDXW)]],
                              rows.at[pl.ds(j * IDXW, IDXW)], sem_g).wait()


def _compute(bufmk, rows, out_v):
    def group_body(pg, c):
        for q in range(0, L, 2):
            mm = bufmk[pl.ds(pg * IDXW + q * M, L)]
            mmf = mm.astype(jnp.float32)
            for t in range(2):
                p = pg * L + q + t
                r = p * M
                ms = [mmf[t * M + m] for m in range(M)]
                lo = rows[r, pl.ds(0, L)] * ms[0]
                hi = rows[r, pl.ds(L, L)] * ms[0]
                for m in range(1, M):
                    lo = lo + rows[r + m, pl.ds(0, L)] * ms[m]
                    hi = hi + rows[r + m, pl.ds(L, L)] * ms[m]
                cnt = ms[0]
                for m in range(1, M):
                    cnt = cnt + ms[m]
                a = jnp.float32(1.0)
                for k in range(2, M + 1):
                    a = jnp.where(cnt == jnp.float32(k),
                                  jnp.float32(1.0 / k), a)
                out_v[p, pl.ds(0, L)] = lo * a
                out_v[p, pl.ds(L, L)] = hi * a
        return c
    lax.fori_loop(0, C // L, group_body, 0)


def _body(ids_hbm, mask_hbm, table_hbm, out_hbm,
          bufid0, bufid1, bufmk0, bufmk1, rows0, rows1, outv0, outv1,
          sem_in0, sem_in1, sem_g0, sem_g1, sem_o0, sem_o1):
    w = lax.axis_index("s") * NC + lax.axis_index("c")
    gc0 = w * NCHUNK

    _stage(ids_hbm, mask_hbm, gc0, bufid0, bufmk0, sem_in0)
    _stage_wait(ids_hbm, mask_hbm, gc0, bufid0, bufmk0, sem_in0)
    _fire_gathers(table_hbm, bufid0, rows0, sem_g0)
    _stage(ids_hbm, mask_hbm, gc0 + 1, bufid1, bufmk1, sem_in1)

    def pair_body(g, carry):
        ca = gc0 + 2 * g
        cb = ca + 1

        _stage_wait(ids_hbm, mask_hbm, cb, bufid1, bufmk1, sem_in1)
        _fire_gathers(table_hbm, bufid1, rows1, sem_g1)

        _drain_gathers(table_hbm, bufid0, rows0, sem_g0)

        @pl.when(g > 0)
        def _():
            pltpu.make_async_copy(outv0, out_hbm.at[pl.ds(0, C)], sem_o0).wait()
        _compute(bufmk0, rows0, outv0)
        pltpu.async_copy(outv0, out_hbm.at[pl.ds(ca * C, C)], sem_o0)

        @pl.when(g < NCHUNK // 2 - 1)
        def _():
            _stage(ids_hbm, mask_hbm, ca + 2, bufid0, bufmk0, sem_in0)
            _stage_wait(ids_hbm, mask_hbm, ca + 2, bufid0, bufmk0, sem_in0)
            _fire_gathers(table_hbm, bufid0, rows0, sem_g0)

        _drain_gathers(table_hbm, bufid1, rows1, sem_g1)

        @pl.when(g > 0)
        def _():
            pltpu.make_async_copy(outv1, out_hbm.at[pl.ds(0, C)], sem_o1).wait()
        _compute(bufmk1, rows1, outv1)
        pltpu.async_copy(outv1, out_hbm.at[pl.ds(cb * C, C)], sem_o1)

        @pl.when(g < NCHUNK // 2 - 1)
        def _():
            _stage(ids_hbm, mask_hbm, cb + 2, bufid1, bufmk1, sem_in1)
        return carry

    lax.fori_loop(0, NCHUNK // 2, pair_body, 0)

    pltpu.make_async_copy(outv0, out_hbm.at[pl.ds(0, C)], sem_o0).wait()
    pltpu.make_async_copy(outv1, out_hbm.at[pl.ds(0, C)], sem_o1).wait()


def kernel(category_ids, category_mask, embedding_table):
    nchunks = P // C
    ids_c = category_ids.reshape(nchunks, C * M).astype(jnp.int32)
    mask_c = category_mask.reshape(nchunks, C * M).astype(jnp.int32)

    mesh = plsc.VectorSubcoreMesh(core_axis_name="c", subcore_axis_name="s",
                                  num_cores=NC, num_subcores=NS)
    out = pl.kernel(
        _body,
        out_type=jax.ShapeDtypeStruct((P, D), jnp.float32),
        mesh=mesh,
        compiler_params=pltpu.CompilerParams(use_tc_tiling_on_sc=False),
        scratch_types=[
            pltpu.VMEM((C * M,), jnp.int32),
            pltpu.VMEM((C * M,), jnp.int32),
            pltpu.VMEM((C * M,), jnp.int32),
            pltpu.VMEM((C * M,), jnp.int32),
            pltpu.VMEM((RC, D), jnp.float32),
            pltpu.VMEM((RC, D), jnp.float32),
            pltpu.VMEM((C, D), jnp.float32),
            pltpu.VMEM((C, D), jnp.float32),
            pltpu.SemaphoreType.DMA,
            pltpu.SemaphoreType.DMA,
            pltpu.SemaphoreType.DMA,
            pltpu.SemaphoreType.DMA,
            pltpu.SemaphoreType.DMA,
            pltpu.SemaphoreType.DMA,
        ],
    )(ids_c, mask_c, embedding_table)
    return out.reshape(category_ids.shape[0], category_ids.shape[1], D)


# speedup vs baseline: 1.3999x; 1.3974x over previous
import jax
import jax.numpy as jnp
from jax import lax
from jax.experimental import pallas as pl
from jax.experimental.pallas import tpu as pltpu
from jax.experimental.pallas import tpu_sc as plsc

NC = 2
NS = 16
L = 16
NW = NC * NS

B = 4096
S = 50
M = 8
D = 32
C = B // NW
RC = C * M


def _stage(ids_hbm, mask_hbm, s, wb, bufid, bufmk, sem_in):
    pltpu.async_copy(ids_hbm.at[s, :, pl.ds(wb, C)], bufid, sem_in)
    pltpu.async_copy(mask_hbm.at[s, :, pl.ds(wb, C)], bufmk, sem_in)


def _stage_wait(ids_hbm, mask_hbm, s, wb, bufid, bufmk, sem_in):
    pltpu.make_async_copy(ids_hbm.at[s, :, pl.ds(wb, C)], bufid, sem_in).wait()
    pltpu.make_async_copy(mask_hbm.at[s, :, pl.ds(wb, C)], bufmk, sem_in).wait()


def _fire_gathers(table_hbm, bufid, rows, sem_g):
    for m in range(M):
        pltpu.async_copy(table_hbm.at[bufid.at[m]],
                         rows.at[pl.ds(m * C, C)], sem_g)


def _drain_gathers(table_hbm, bufid, rows, sem_g):
    for m in range(M):
        pltpu.make_async_copy(table_hbm.at[bufid.at[m]],
                              rows.at[pl.ds(m * C, C)], sem_g).wait()


def _compute(bufmk, rows, out_v):
    def group_body(bg, c):
        bs = pl.ds(bg * L, L)
        mmf = [bufmk[m, bs].astype(jnp.float32) for m in range(M)]
        cnt = mmf[0]
        for m in range(1, M):
            cnt = cnt + mmf[m]
        a16 = 1.0 / jnp.maximum(cnt, 1.0)
        for t in range(L):
            b = bg * L + t
            lo = rows[b, pl.ds(0, L)] * mmf[0][t]
            hi = rows[b, pl.ds(L, L)] * mmf[0][t]
            for m in range(1, M):
                lo = lo + rows[m * C + b, pl.ds(0, L)] * mmf[m][t]
                hi = hi + rows[m * C + b, pl.ds(L, L)] * mmf[m][t]
            a = a16[t]
            out_v[b, pl.ds(0, L)] = lo * a
            out_v[b, pl.ds(L, L)] = hi * a
        return c
    lax.fori_loop(0, C // L, group_body, 0)


def _body(ids_hbm, mask_hbm, table_hbm, out_hbm,
          bufid0, bufid1, bufmk0, bufmk1, rows0, rows1, outv0, outv1,
          sem_in0, sem_in1, sem_g0, sem_g1, sem_o0, sem_o1):
    w = lax.axis_index("s") * NC + lax.axis_index("c")
    wb = w * C

    _stage(ids_hbm, mask_hbm, 0, wb, bufid0, bufmk0, sem_in0)
    _stage_wait(ids_hbm, mask_hbm, 0, wb, bufid0, bufmk0, sem_in0)
    _fire_gathers(table_hbm, bufid0, rows0, sem_g0)
    _stage(ids_hbm, mask_hbm, 1, wb, bufid1, bufmk1, sem_in1)

    def pair_body(g, carry):
        sa = 2 * g
        sb = sa + 1

        _stage_wait(ids_hbm, mask_hbm, sb, wb, bufid1, bufmk1, sem_in1)
        _fire_gathers(table_hbm, bufid1, rows1, sem_g1)

        _drain_gathers(table_hbm, bufid0, rows0, sem_g0)

        @pl.when(g > 0)
        def _():
            pltpu.make_async_copy(outv0, out_hbm.at[pl.ds(wb, C), 0],
                                  sem_o0).wait()
        _compute(bufmk0, rows0, outv0)
        pltpu.async_copy(outv0, out_hbm.at[pl.ds(wb, C), sa], sem_o0)

        @pl.when(g < S // 2 - 1)
        def _():
            _stage(ids_hbm, mask_hbm, sa + 2, wb, bufid0, bufmk0, sem_in0)
            _stage_wait(ids_hbm, mask_hbm, sa + 2, wb, bufid0, bufmk0, sem_in0)
            _fire_gathers(table_hbm, bufid0, rows0, sem_g0)

        _drain_gathers(table_hbm, bufid1, rows1, sem_g1)

        @pl.when(g > 0)
        def _():
            pltpu.make_async_copy(outv1, out_hbm.at[pl.ds(wb, C), 0],
                                  sem_o1).wait()
        _compute(bufmk1, rows1, outv1)
        pltpu.async_copy(outv1, out_hbm.at[pl.ds(wb, C), sb], sem_o1)

        @pl.when(g < S // 2 - 1)
        def _():
            _stage(ids_hbm, mask_hbm, sb + 2, wb, bufid1, bufmk1, sem_in1)
        return carry

    lax.fori_loop(0, S // 2, pair_body, 0)

    pltpu.make_async_copy(outv0, out_hbm.at[pl.ds(wb, C), 0], sem_o0).wait()
    pltpu.make_async_copy(outv1, out_hbm.at[pl.ds(wb, C), 0], sem_o1).wait()


def kernel(category_ids, category_mask, embedding_table):
    ids_t = jnp.transpose(category_ids.astype(jnp.int32), (1, 2, 0))
    mask_t = jnp.transpose(category_mask.astype(jnp.int32), (1, 2, 0))

    mesh = plsc.VectorSubcoreMesh(core_axis_name="c", subcore_axis_name="s",
                                  num_cores=NC, num_subcores=NS)
    out = pl.kernel(
        _body,
        out_type=jax.ShapeDtypeStruct((B, S, D), jnp.float32),
        mesh=mesh,
        compiler_params=pltpu.CompilerParams(use_tc_tiling_on_sc=False),
        scratch_types=[
            pltpu.VMEM((M, C), jnp.int32),
            pltpu.VMEM((M, C), jnp.int32),
            pltpu.VMEM((M, C), jnp.int32),
            pltpu.VMEM((M, C), jnp.int32),
            pltpu.VMEM((RC, D), jnp.float32),
            pltpu.VMEM((RC, D), jnp.float32),
            pltpu.VMEM((C, D), jnp.float32),
            pltpu.VMEM((C, D), jnp.float32),
            pltpu.SemaphoreType.DMA,
            pltpu.SemaphoreType.DMA,
            pltpu.SemaphoreType.DMA,
            pltpu.SemaphoreType.DMA,
            pltpu.SemaphoreType.DMA,
            pltpu.SemaphoreType.DMA,
        ],
    )(ids_t, mask_t, embedding_table)
    return out
